# Initial kernel scaffold; baseline (speedup 1.0000x reference)
import jax
import jax.numpy as jnp
from jax import lax
from jax.experimental import pallas as pl
from jax.experimental.pallas import tpu as pltpu
from jax.experimental.pallas import tpu_sc as plsc

N_NODE = 10000
D = 128
E = 160000
NC, NS = 2, 16
K = 40
CPT = E // (NC * NS * K)
ROWS_PER_TILE = N_NODE // NS


def _sc_segsum_body(xp_hbm, xn_hbm, srcp, dstp, srcn, dstn, out_hbm,
                    zbuf, isrc, idst, rbuf, acc):
    cid = lax.axis_index("c")
    sid = lax.axis_index("s")

    def _z(i, _):
        r = i // 8
        cb = i % 8
        zbuf[r, pl.ds(cb * 16, 16)] = jnp.zeros((16,), jnp.float32)
        return 0
    lax.fori_loop(0, 25 * 8, _z, 0)

    def _zacc(i, _):
        pltpu.sync_copy(zbuf, acc.at[pl.ds(sid * ROWS_PER_TILE + i * 25, 25)])
        return 0
    lax.fori_loop(0, ROWS_PER_TILE // 25, _zacc, 0)
    plsc.subcore_barrier()

    base = (cid * NS + sid) * CPT
    for src_ref, dst_ref, tab_ref in ((srcp, dstp, xp_hbm),
                                      (srcn, dstn, xn_hbm)):
        pltpu.sync_copy(src_ref.at[pl.ds(base, CPT)], isrc)
        pltpu.sync_copy(dst_ref.at[pl.ds(base, CPT)], idst)

        def _chunk(j, _):
            pltpu.sync_copy(tab_ref.at[isrc.at[j]], rbuf)
            pltpu.sync_copy(rbuf, acc.at[idst.at[j]], add=True)
            return 0
        lax.fori_loop(0, CPT, _chunk, 0)

    plsc.subcore_barrier()
    pltpu.sync_copy(acc.at[pl.ds(sid * ROWS_PER_TILE, ROWS_PER_TILE)],
                    out_hbm.at[cid, pl.ds(sid * ROWS_PER_TILE, ROWS_PER_TILE)])


def _make_sc_segsum(interpret=False):
    mesh = plsc.VectorSubcoreMesh(core_axis_name="c", subcore_axis_name="s")
    return pl.kernel(
        _sc_segsum_body,
        out_type=jax.ShapeDtypeStruct((NC, N_NODE, D), jnp.float32),
        mesh=mesh,
        scratch_types=[
            pltpu.VMEM((25, D), jnp.float32),
            pltpu.VMEM((CPT, K), jnp.int32),
            pltpu.VMEM((CPT, K), jnp.int32),
            pltpu.VMEM((K, D), jnp.float32),
            pltpu.VMEM_SHARED((N_NODE, D), jnp.float32),
        ],
        interpret=interpret,
    )


ROWS_BLK = 2000


def _tc_proj_body(x_ref, w1_ref, w2_ref, p1_ref, p2_ref):
    x = x_ref[...]
    p1_ref[...] = jnp.dot(x, w1_ref[...], preferred_element_type=jnp.float32)
    p2_ref[...] = jnp.dot(x, w2_ref[...], preferred_element_type=jnp.float32)


def _make_tc_proj(interpret=False):
    return pl.pallas_call(
        _tc_proj_body,
        grid=(N_NODE // ROWS_BLK,),
        in_specs=[
            pl.BlockSpec((ROWS_BLK, D), lambda i: (i, 0)),
            pl.BlockSpec((D, D), lambda i: (0, 0)),
            pl.BlockSpec((D, D), lambda i: (0, 0)),
        ],
        out_specs=[
            pl.BlockSpec((ROWS_BLK, D), lambda i: (i, 0)),
            pl.BlockSpec((ROWS_BLK, D), lambda i: (i, 0)),
        ],
        out_shape=[jax.ShapeDtypeStruct((N_NODE, D), jnp.float32)] * 2,
        interpret=interpret,
    )


def _tc_update_proj_body(x_ref, a_ref, w1_ref, w2_ref,
                         xn_ref, p1_ref, p2_ref):
    xn = jax.nn.relu(x_ref[...] + a_ref[0] + a_ref[1])
    xn_ref[...] = xn
    p1_ref[...] = jnp.dot(xn, w1_ref[...], preferred_element_type=jnp.float32)
    p2_ref[...] = jnp.dot(xn, w2_ref[...], preferred_element_type=jnp.float32)


def _make_tc_update_proj(interpret=False):
    return pl.pallas_call(
        _tc_update_proj_body,
        grid=(N_NODE // ROWS_BLK,),
        in_specs=[
            pl.BlockSpec((ROWS_BLK, D), lambda i: (i, 0)),
            pl.BlockSpec((NC, ROWS_BLK, D), lambda i: (0, i, 0)),
            pl.BlockSpec((D, D), lambda i: (0, 0)),
            pl.BlockSpec((D, D), lambda i: (0, 0)),
        ],
        out_specs=[
            pl.BlockSpec((ROWS_BLK, D), lambda i: (i, 0)),
            pl.BlockSpec((ROWS_BLK, D), lambda i: (i, 0)),
            pl.BlockSpec((ROWS_BLK, D), lambda i: (i, 0)),
        ],
        out_shape=[jax.ShapeDtypeStruct((N_NODE, D), jnp.float32)] * 3,
        interpret=interpret,
    )


def _layer_norm(x, g, b):
    mu = jnp.mean(x, axis=-1, keepdims=True)
    var = jnp.mean((x - mu) * (x - mu), axis=-1, keepdims=True)
    return (x - mu) * lax.rsqrt(var + 1e-6) * g + b


def _tc_final_body(xv_ref, a_ref, xc_ref, g_ref, b_ref, yv_ref, yc_ref):
    g = g_ref[...]
    b = b_ref[...]
    xvn = jax.nn.relu(xv_ref[...] + a_ref[0] + a_ref[1])
    yv_ref[...] = _layer_norm(xvn, g, b)
    yc_ref[...] = _layer_norm(xc_ref[...], g, b)


def _make_tc_final(interpret=False):
    return pl.pallas_call(
        _tc_final_body,
        grid=(N_NODE // ROWS_BLK,),
        in_specs=[
            pl.BlockSpec((ROWS_BLK, D), lambda i: (i, 0)),
            pl.BlockSpec((NC, ROWS_BLK, D), lambda i: (0, i, 0)),
            pl.BlockSpec((ROWS_BLK, D), lambda i: (i, 0)),
            pl.BlockSpec((1, D), lambda i: (0, 0)),
            pl.BlockSpec((1, D), lambda i: (0, 0)),
        ],
        out_specs=[
            pl.BlockSpec((ROWS_BLK, D), lambda i: (i, 0)),
            pl.BlockSpec((ROWS_BLK, D), lambda i: (i, 0)),
        ],
        out_shape=[jax.ShapeDtypeStruct((N_NODE, D), jnp.float32)] * 2,
        interpret=interpret,
    )


def _impl(xv, xc, adj_pos, adj_neg, Wcp, Wcn, Wvp, Wvn, gamma, beta,
          interpret=False):
    sc_segsum = _make_sc_segsum(interpret)
    tc_proj = _make_tc_proj(interpret)
    tc_update = _make_tc_update_proj(interpret)
    tc_final = _make_tc_final(interpret)

    cp = adj_pos[0].astype(jnp.int32).reshape(E // K, K)
    lp = adj_pos[1].astype(jnp.int32).reshape(E // K, K)
    cn = adj_neg[0].astype(jnp.int32).reshape(E // K, K)
    ln_ = adj_neg[1].astype(jnp.int32).reshape(E // K, K)
    g2 = gamma.reshape(1, D)
    b2 = beta.reshape(1, D)

    L = Wcp.shape[0]
    xvp, xvn = tc_proj(xv, Wcp[0], Wcn[0])
    acc_c = sc_segsum(xvp, xvn, lp, cp, ln_, cn)
    xc, xcp, xcn = tc_update(xc, acc_c, Wvp[0], Wvn[0])
    acc_v = sc_segsum(xcp, xcn, cp, lp, cn, ln_)
    for l in range(1, L):
        xv, xvp, xvn = tc_update(xv, acc_v, Wcp[l], Wcn[l])
        acc_c = sc_segsum(xvp, xvn, lp, cp, ln_, cn)
        xc, xcp, xcn = tc_update(xc, acc_c, Wvp[l], Wvn[l])
        acc_v = sc_segsum(xcp, xcn, cp, lp, cn, ln_)
    return tc_final(xv, acc_v, xc, g2, b2)


def kernel(xv, xc, adj_pos, adj_neg, Wcp, Wcn, Wvp, Wvn, gamma, beta):
    yv, yc = _impl(xv, xc, adj_pos, adj_neg, Wcp, Wcn, Wvp, Wvn, gamma, beta)
    return yv, yc


# speedup vs baseline: 4.2454x; 4.2454x over previous
import jax
import jax.numpy as jnp
from jax import lax
from jax.experimental import pallas as pl
from jax.experimental.pallas import tpu as pltpu
from jax.experimental.pallas import tpu_sc as plsc

N_NODE = 10000
D = 128
E = 160000
NC, NS = 2, 16
K = 40
CPT = E // (NC * NS * K)
ROWS_PER_TILE = 632
N_PAD = NS * ROWS_PER_TILE


def _sc_segsum_body(xp_hbm, xn_hbm, srcp, dstp, srcn, dstn, out_hbm,
                    zbuf, isrc, idst, rbuf, acc):
    cid = lax.axis_index("c")
    sid = lax.axis_index("s")

    def _z(i, _):
        r = i // 8
        cb = i % 8
        zbuf[r, pl.ds(cb * 16, 16)] = jnp.zeros((16,), jnp.float32)
        return 0
    lax.fori_loop(0, 8 * 8, _z, 0)

    def _zacc(i, _):
        pltpu.sync_copy(zbuf, acc.at[pl.ds(sid * ROWS_PER_TILE + i * 8, 8)])
        return 0
    lax.fori_loop(0, ROWS_PER_TILE // 8, _zacc, 0)
    plsc.subcore_barrier()

    wid = cid * NS + sid
    for src_ref, dst_ref, tab_ref in ((srcp, dstp, xp_hbm),
                                      (srcn, dstn, xn_hbm)):
        pltpu.sync_copy(src_ref.at[wid], isrc)
        pltpu.sync_copy(dst_ref.at[wid], idst)

        def _chunk(j, _):
            pltpu.sync_copy(tab_ref.at[isrc.at[j]], rbuf)
            pltpu.sync_copy(rbuf, acc.at[idst.at[j]], add=True)
            return 0
        lax.fori_loop(0, CPT, _chunk, 0)

    plsc.subcore_barrier()
    pltpu.sync_copy(acc.at[pl.ds(sid * ROWS_PER_TILE, ROWS_PER_TILE)],
                    out_hbm.at[cid, pl.ds(sid * ROWS_PER_TILE, ROWS_PER_TILE)])


def _make_sc_segsum(interpret=False):
    mesh = plsc.VectorSubcoreMesh(core_axis_name="c", subcore_axis_name="s",
                                  num_cores=NC, num_subcores=NS)
    return pl.kernel(
        _sc_segsum_body,
        out_type=jax.ShapeDtypeStruct((NC, N_PAD, D), jnp.float32),
        mesh=mesh,
        scratch_types=[
            pltpu.VMEM((8, D), jnp.float32),
            pltpu.VMEM((CPT, K), jnp.int32),
            pltpu.VMEM((CPT, K), jnp.int32),
            pltpu.VMEM((K, D), jnp.float32),
            pltpu.VMEM_SHARED((N_PAD, D), jnp.float32),
        ],
        interpret=interpret,
    )


ROWS_BLK = 2000


def _tc_proj_body(x_ref, w1_ref, w2_ref, p1_ref, p2_ref):
    x = x_ref[...]
    p1_ref[...] = jnp.dot(x, w1_ref[...], preferred_element_type=jnp.float32)
    p2_ref[...] = jnp.dot(x, w2_ref[...], preferred_element_type=jnp.float32)


def _make_tc_proj(interpret=False):
    return pl.pallas_call(
        _tc_proj_body,
        grid=(N_NODE // ROWS_BLK,),
        in_specs=[
            pl.BlockSpec((ROWS_BLK, D), lambda i: (i, 0)),
            pl.BlockSpec((D, D), lambda i: (0, 0)),
            pl.BlockSpec((D, D), lambda i: (0, 0)),
        ],
        out_specs=[
            pl.BlockSpec((ROWS_BLK, D), lambda i: (i, 0)),
            pl.BlockSpec((ROWS_BLK, D), lambda i: (i, 0)),
        ],
        out_shape=[jax.ShapeDtypeStruct((N_NODE, D), jnp.float32)] * 2,
        interpret=interpret,
    )


def _tc_update_proj_body(x_ref, a_ref, w1_ref, w2_ref,
                         xn_ref, p1_ref, p2_ref):
    xn = jax.nn.relu(x_ref[...] + a_ref[0] + a_ref[1])
    xn_ref[...] = xn
    p1_ref[...] = jnp.dot(xn, w1_ref[...], preferred_element_type=jnp.float32)
    p2_ref[...] = jnp.dot(xn, w2_ref[...], preferred_element_type=jnp.float32)


def _make_tc_update_proj(interpret=False):
    return pl.pallas_call(
        _tc_update_proj_body,
        grid=(N_NODE // ROWS_BLK,),
        in_specs=[
            pl.BlockSpec((ROWS_BLK, D), lambda i: (i, 0)),
            pl.BlockSpec((NC, ROWS_BLK, D), lambda i: (0, i, 0)),
            pl.BlockSpec((D, D), lambda i: (0, 0)),
            pl.BlockSpec((D, D), lambda i: (0, 0)),
        ],
        out_specs=[
            pl.BlockSpec((ROWS_BLK, D), lambda i: (i, 0)),
            pl.BlockSpec((ROWS_BLK, D), lambda i: (i, 0)),
            pl.BlockSpec((ROWS_BLK, D), lambda i: (i, 0)),
        ],
        out_shape=[jax.ShapeDtypeStruct((N_NODE, D), jnp.float32)] * 3,
        interpret=interpret,
    )


def _layer_norm(x, g, b):
    mu = jnp.mean(x, axis=-1, keepdims=True)
    var = jnp.mean((x - mu) * (x - mu), axis=-1, keepdims=True)
    return (x - mu) * lax.rsqrt(var + 1e-6) * g + b


def _tc_final_body(xv_ref, a_ref, xc_ref, g_ref, b_ref, yv_ref, yc_ref):
    g = g_ref[...]
    b = b_ref[...]
    xvn = jax.nn.relu(xv_ref[...] + a_ref[0] + a_ref[1])
    yv_ref[...] = _layer_norm(xvn, g, b)
    yc_ref[...] = _layer_norm(xc_ref[...], g, b)


def _make_tc_final(interpret=False):
    return pl.pallas_call(
        _tc_final_body,
        grid=(N_NODE // ROWS_BLK,),
        in_specs=[
            pl.BlockSpec((ROWS_BLK, D), lambda i: (i, 0)),
            pl.BlockSpec((NC, ROWS_BLK, D), lambda i: (0, i, 0)),
            pl.BlockSpec((ROWS_BLK, D), lambda i: (i, 0)),
            pl.BlockSpec((1, D), lambda i: (0, 0)),
            pl.BlockSpec((1, D), lambda i: (0, 0)),
        ],
        out_specs=[
            pl.BlockSpec((ROWS_BLK, D), lambda i: (i, 0)),
            pl.BlockSpec((ROWS_BLK, D), lambda i: (i, 0)),
        ],
        out_shape=[jax.ShapeDtypeStruct((N_NODE, D), jnp.float32)] * 2,
        interpret=interpret,
    )


def _impl(xv, xc, adj_pos, adj_neg, Wcp, Wcn, Wvp, Wvn, gamma, beta,
          interpret=False):
    sc_segsum = _make_sc_segsum(interpret)
    tc_proj = _make_tc_proj(interpret)
    tc_update = _make_tc_update_proj(interpret)
    tc_final = _make_tc_final(interpret)

    idx_shape = (NC * NS, CPT, K)
    cp = adj_pos[0].astype(jnp.int32).reshape(idx_shape)
    lp = adj_pos[1].astype(jnp.int32).reshape(idx_shape)
    cn = adj_neg[0].astype(jnp.int32).reshape(idx_shape)
    ln_ = adj_neg[1].astype(jnp.int32).reshape(idx_shape)
    g2 = gamma.reshape(1, D)
    b2 = beta.reshape(1, D)

    L = Wcp.shape[0]
    xvp, xvn = tc_proj(xv, Wcp[0], Wcn[0])
    acc_c = sc_segsum(xvp, xvn, lp, cp, ln_, cn)
    xc, xcp, xcn = tc_update(xc, acc_c, Wvp[0], Wvn[0])
    acc_v = sc_segsum(xcp, xcn, cp, lp, cn, ln_)
    for l in range(1, L):
        xv, xvp, xvn = tc_update(xv, acc_v, Wcp[l], Wcn[l])
        acc_c = sc_segsum(xvp, xvn, lp, cp, ln_, cn)
        xc, xcp, xcn = tc_update(xc, acc_c, Wvp[l], Wvn[l])
        acc_v = sc_segsum(xcp, xcn, cp, lp, cn, ln_)
    return tc_final(xv, acc_v, xc, g2, b2)


def kernel(xv, xc, adj_pos, adj_neg, Wcp, Wcn, Wvp, Wvn, gamma, beta):
    yv, yc = _impl(xv, xc, adj_pos, adj_neg, Wcp, Wcn, Wvp, Wvn, gamma, beta)
    return yv, yc


# speedup vs baseline: 8.1742x; 1.9254x over previous
import jax
import jax.numpy as jnp
from jax import lax
from jax.experimental import pallas as pl
from jax.experimental.pallas import tpu as pltpu
from jax.experimental.pallas import tpu_sc as plsc

N_NODE = 10000
D = 128
E = 160000
NC, NS = 2, 16
K = 100
CPT = E // (NC * NS * K)
ROWS_PER_TILE = 632
N_PAD = NS * ROWS_PER_TILE


def _sc_segsum_body(xp_hbm, xn_hbm, srcp, dstp, srcn, dstn, out_hbm,
                    zbuf, isrc, idst, rbuf, acc, gsem, ssem):
    cid = lax.axis_index("c")
    sid = lax.axis_index("s")

    def _z(i, _):
        r = i // 8
        cb = i % 8
        zbuf[r, pl.ds(cb * 16, 16)] = jnp.zeros((16,), jnp.float32)
        return 0
    lax.fori_loop(0, 8 * 8, _z, 0)

    def _zacc(i, _):
        pltpu.sync_copy(zbuf, acc.at[pl.ds(sid * ROWS_PER_TILE + i * 8, 8)])
        return 0
    lax.fori_loop(0, ROWS_PER_TILE // 8, _zacc, 0)
    plsc.subcore_barrier()

    wid = cid * NS + sid
    for src_ref, dst_ref, tab_ref in ((srcp, dstp, xp_hbm),
                                      (srcn, dstn, xn_hbm)):
        pltpu.sync_copy(src_ref.at[wid], isrc)
        pltpu.sync_copy(dst_ref.at[wid], idst)

        def _g_start(j, m):
            pltpu.async_copy(tab_ref.at[isrc.at[j]], rbuf.at[m], gsem)

        def _g_wait(j, m):
            pltpu.make_async_copy(tab_ref.at[isrc.at[j]], rbuf.at[m],
                                  gsem).wait()

        def _s_start(j, m):
            pltpu.async_copy(rbuf.at[m], acc.at[idst.at[j]], ssem, add=True)

        def _s_wait(j, m):
            pltpu.make_async_copy(rbuf.at[m], acc.at[idst.at[j]],
                                  ssem).wait()

        _g_start(0, 0)
        _g_wait(0, 0)
        _g_start(1, 1)
        _s_start(0, 0)

        def _body(j, _):
            m = j % 2
            _g_wait(j, m)
            _s_wait(j - 1, 1 - m)
            _g_start(j + 1, 1 - m)
            _s_start(j, m)
            return 0
        lax.fori_loop(1, CPT - 1, _body, 0)

        _g_wait(CPT - 1, (CPT - 1) % 2)
        _s_wait(CPT - 2, CPT % 2)
        _s_start(CPT - 1, (CPT - 1) % 2)
        _s_wait(CPT - 1, (CPT - 1) % 2)

    plsc.subcore_barrier()
    pltpu.sync_copy(acc.at[pl.ds(sid * ROWS_PER_TILE, ROWS_PER_TILE)],
                    out_hbm.at[cid, pl.ds(sid * ROWS_PER_TILE, ROWS_PER_TILE)])


def _make_sc_segsum(interpret=False):
    mesh = plsc.VectorSubcoreMesh(core_axis_name="c", subcore_axis_name="s",
                                  num_cores=NC, num_subcores=NS)
    return pl.kernel(
        _sc_segsum_body,
        out_type=jax.ShapeDtypeStruct((NC, N_PAD, D), jnp.float32),
        mesh=mesh,
        scratch_types=[
            pltpu.VMEM((8, D), jnp.float32),
            pltpu.VMEM((CPT, K), jnp.int32),
            pltpu.VMEM((CPT, K), jnp.int32),
            pltpu.VMEM((2, K, D), jnp.float32),
            pltpu.VMEM_SHARED((N_PAD, D), jnp.float32),
            pltpu.SemaphoreType.DMA,
            pltpu.SemaphoreType.DMA,
        ],
        interpret=interpret,
    )


ROWS_BLK = 2000


def _tc_proj_body(x_ref, w1_ref, w2_ref, p1_ref, p2_ref):
    x = x_ref[...]
    p1_ref[...] = jnp.dot(x, w1_ref[...], preferred_element_type=jnp.float32)
    p2_ref[...] = jnp.dot(x, w2_ref[...], preferred_element_type=jnp.float32)


def _make_tc_proj(interpret=False):
    return pl.pallas_call(
        _tc_proj_body,
        grid=(N_NODE // ROWS_BLK,),
        in_specs=[
            pl.BlockSpec((ROWS_BLK, D), lambda i: (i, 0)),
            pl.BlockSpec((D, D), lambda i: (0, 0)),
            pl.BlockSpec((D, D), lambda i: (0, 0)),
        ],
        out_specs=[
            pl.BlockSpec((ROWS_BLK, D), lambda i: (i, 0)),
            pl.BlockSpec((ROWS_BLK, D), lambda i: (i, 0)),
        ],
        out_shape=[jax.ShapeDtypeStruct((N_NODE, D), jnp.float32)] * 2,
        interpret=interpret,
    )


def _tc_update_proj_body(x_ref, a_ref, w1_ref, w2_ref,
                         xn_ref, p1_ref, p2_ref):
    xn = jax.nn.relu(x_ref[...] + a_ref[0] + a_ref[1])
    xn_ref[...] = xn
    p1_ref[...] = jnp.dot(xn, w1_ref[...], preferred_element_type=jnp.float32)
    p2_ref[...] = jnp.dot(xn, w2_ref[...], preferred_element_type=jnp.float32)


def _make_tc_update_proj(interpret=False):
    return pl.pallas_call(
        _tc_update_proj_body,
        grid=(N_NODE // ROWS_BLK,),
        in_specs=[
            pl.BlockSpec((ROWS_BLK, D), lambda i: (i, 0)),
            pl.BlockSpec((NC, ROWS_BLK, D), lambda i: (0, i, 0)),
            pl.BlockSpec((D, D), lambda i: (0, 0)),
            pl.BlockSpec((D, D), lambda i: (0, 0)),
        ],
        out_specs=[
            pl.BlockSpec((ROWS_BLK, D), lambda i: (i, 0)),
            pl.BlockSpec((ROWS_BLK, D), lambda i: (i, 0)),
            pl.BlockSpec((ROWS_BLK, D), lambda i: (i, 0)),
        ],
        out_shape=[jax.ShapeDtypeStruct((N_NODE, D), jnp.float32)] * 3,
        interpret=interpret,
    )


def _layer_norm(x, g, b):
    mu = jnp.mean(x, axis=-1, keepdims=True)
    var = jnp.mean((x - mu) * (x - mu), axis=-1, keepdims=True)
    return (x - mu) * lax.rsqrt(var + 1e-6) * g + b


def _tc_final_body(xv_ref, a_ref, xc_ref, g_ref, b_ref, yv_ref, yc_ref):
    g = g_ref[...]
    b = b_ref[...]
    xvn = jax.nn.relu(xv_ref[...] + a_ref[0] + a_ref[1])
    yv_ref[...] = _layer_norm(xvn, g, b)
    yc_ref[...] = _layer_norm(xc_ref[...], g, b)


def _make_tc_final(interpret=False):
    return pl.pallas_call(
        _tc_final_body,
        grid=(N_NODE // ROWS_BLK,),
        in_specs=[
            pl.BlockSpec((ROWS_BLK, D), lambda i: (i, 0)),
            pl.BlockSpec((NC, ROWS_BLK, D), lambda i: (0, i, 0)),
            pl.BlockSpec((ROWS_BLK, D), lambda i: (i, 0)),
            pl.BlockSpec((1, D), lambda i: (0, 0)),
            pl.BlockSpec((1, D), lambda i: (0, 0)),
        ],
        out_specs=[
            pl.BlockSpec((ROWS_BLK, D), lambda i: (i, 0)),
            pl.BlockSpec((ROWS_BLK, D), lambda i: (i, 0)),
        ],
        out_shape=[jax.ShapeDtypeStruct((N_NODE, D), jnp.float32)] * 2,
        interpret=interpret,
    )


def _impl(xv, xc, adj_pos, adj_neg, Wcp, Wcn, Wvp, Wvn, gamma, beta,
          interpret=False):
    sc_segsum = _make_sc_segsum(interpret)
    tc_proj = _make_tc_proj(interpret)
    tc_update = _make_tc_update_proj(interpret)
    tc_final = _make_tc_final(interpret)

    idx_shape = (NC * NS, CPT, K)
    cp = adj_pos[0].astype(jnp.int32).reshape(idx_shape)
    lp = adj_pos[1].astype(jnp.int32).reshape(idx_shape)
    cn = adj_neg[0].astype(jnp.int32).reshape(idx_shape)
    ln_ = adj_neg[1].astype(jnp.int32).reshape(idx_shape)
    g2 = gamma.reshape(1, D)
    b2 = beta.reshape(1, D)

    L = Wcp.shape[0]
    xvp, xvn = tc_proj(xv, Wcp[0], Wcn[0])
    acc_c = sc_segsum(xvp, xvn, lp, cp, ln_, cn)
    xc, xcp, xcn = tc_update(xc, acc_c, Wvp[0], Wvn[0])
    acc_v = sc_segsum(xcp, xcn, cp, lp, cn, ln_)
    for l in range(1, L):
        xv, xvp, xvn = tc_update(xv, acc_v, Wcp[l], Wcn[l])
        acc_c = sc_segsum(xvp, xvn, lp, cp, ln_, cn)
        xc, xcp, xcn = tc_update(xc, acc_c, Wvp[l], Wvn[l])
        acc_v = sc_segsum(xcp, xcn, cp, lp, cn, ln_)
    return tc_final(xv, acc_v, xc, g2, b2)


def kernel(xv, xc, adj_pos, adj_neg, Wcp, Wcn, Wvp, Wvn, gamma, beta):
    yv, yc = _impl(xv, xc, adj_pos, adj_neg, Wcp, Wcn, Wvp, Wvn, gamma, beta)
    return yv, yc


# speedup vs baseline: 8.7904x; 1.0754x over previous
import jax
import jax.numpy as jnp
from jax import lax
from jax.experimental import pallas as pl
from jax.experimental.pallas import tpu as pltpu
from jax.experimental.pallas import tpu_sc as plsc

N_NODE = 10000
D = 128
E = 160000
NC, NS = 2, 16
K = 125
NBUF = 2
CPT = E // (NC * NS * K)
ROWS_PER_TILE = 632
N_PAD = NS * ROWS_PER_TILE


def _sc_segsum_body(xp_hbm, xn_hbm, srcp, dstp, srcn, dstn, out_hbm,
                    zbuf, isrc, idst, rbuf, acc, gsem, ssem):
    cid = lax.axis_index("c")
    sid = lax.axis_index("s")

    def _z(i, _):
        r = i // 8
        cb = i % 8
        zbuf[r, pl.ds(cb * 16, 16)] = jnp.zeros((16,), jnp.float32)
        return 0
    lax.fori_loop(0, 8 * 8, _z, 0)

    def _zacc(i, _):
        pltpu.sync_copy(zbuf, acc.at[pl.ds(sid * ROWS_PER_TILE + i * 8, 8)])
        return 0
    lax.fori_loop(0, ROWS_PER_TILE // 8, _zacc, 0)
    plsc.subcore_barrier()

    wid = cid * NS + sid
    for src_ref, dst_ref, tab_ref in ((srcp, dstp, xp_hbm),
                                      (srcn, dstn, xn_hbm)):
        pltpu.sync_copy(src_ref.at[wid], isrc)
        pltpu.sync_copy(dst_ref.at[wid], idst)

        def _g_start(j, m):
            pltpu.async_copy(tab_ref.at[isrc.at[j]], rbuf.at[m], gsem)

        def _g_wait(j, m):
            pltpu.make_async_copy(tab_ref.at[isrc.at[j]], rbuf.at[m],
                                  gsem).wait()

        def _s_start(j, m):
            pltpu.async_copy(rbuf.at[m], acc.at[idst.at[j]], ssem, add=True)

        def _s_wait(j, m):
            pltpu.make_async_copy(rbuf.at[m], acc.at[idst.at[j]],
                                  ssem).wait()

        for t in range(NBUF - 1):
            _g_start(t, t)
        _g_wait(0, 0)
        _s_start(0, 0)
        _g_start(NBUF - 1, NBUF - 1)

        def _body(j, _):
            m = j % NBUF
            _g_wait(j, m)
            _s_wait(j - 1, (j - 1) % NBUF)
            _g_start(j + NBUF - 1, (j - 1) % NBUF)
            _s_start(j, m)
            return 0
        lax.fori_loop(1, CPT - NBUF + 1, _body, 0)

        for j in range(CPT - NBUF + 1, CPT):
            _g_wait(j, j % NBUF)
            _s_wait(j - 1, (j - 1) % NBUF)
            _s_start(j, j % NBUF)
        _s_wait(CPT - 1, (CPT - 1) % NBUF)

    plsc.subcore_barrier()
    pltpu.sync_copy(acc.at[pl.ds(sid * ROWS_PER_TILE, ROWS_PER_TILE)],
                    out_hbm.at[cid, pl.ds(sid * ROWS_PER_TILE, ROWS_PER_TILE)])


def _make_sc_segsum(interpret=False):
    mesh = plsc.VectorSubcoreMesh(core_axis_name="c", subcore_axis_name="s",
                                  num_cores=NC, num_subcores=NS)
    return pl.kernel(
        _sc_segsum_body,
        out_type=jax.ShapeDtypeStruct((NC, N_PAD, D), jnp.float32),
        mesh=mesh,
        scratch_types=[
            pltpu.VMEM((8, D), jnp.float32),
            pltpu.VMEM((CPT, K), jnp.int32),
            pltpu.VMEM((CPT, K), jnp.int32),
            pltpu.VMEM((NBUF, K, D), jnp.float32),
            pltpu.VMEM_SHARED((N_PAD, D), jnp.float32),
            pltpu.SemaphoreType.DMA,
            pltpu.SemaphoreType.DMA,
        ],
        interpret=interpret,
    )


ROWS_BLK = 2000


def _tc_proj_body(x_ref, w1_ref, w2_ref, p1_ref, p2_ref):
    x = x_ref[...]
    p1_ref[...] = jnp.dot(x, w1_ref[...], preferred_element_type=jnp.float32)
    p2_ref[...] = jnp.dot(x, w2_ref[...], preferred_element_type=jnp.float32)


def _make_tc_proj(interpret=False):
    return pl.pallas_call(
        _tc_proj_body,
        grid=(N_NODE // ROWS_BLK,),
        in_specs=[
            pl.BlockSpec((ROWS_BLK, D), lambda i: (i, 0)),
            pl.BlockSpec((D, D), lambda i: (0, 0)),
            pl.BlockSpec((D, D), lambda i: (0, 0)),
        ],
        out_specs=[
            pl.BlockSpec((ROWS_BLK, D), lambda i: (i, 0)),
            pl.BlockSpec((ROWS_BLK, D), lambda i: (i, 0)),
        ],
        out_shape=[jax.ShapeDtypeStruct((N_NODE, D), jnp.float32)] * 2,
        interpret=interpret,
    )


def _tc_update_proj_body(x_ref, a_ref, w1_ref, w2_ref,
                         xn_ref, p1_ref, p2_ref):
    xn = jax.nn.relu(x_ref[...] + a_ref[0] + a_ref[1])
    xn_ref[...] = xn
    p1_ref[...] = jnp.dot(xn, w1_ref[...], preferred_element_type=jnp.float32)
    p2_ref[...] = jnp.dot(xn, w2_ref[...], preferred_element_type=jnp.float32)


def _make_tc_update_proj(interpret=False):
    return pl.pallas_call(
        _tc_update_proj_body,
        grid=(N_NODE // ROWS_BLK,),
        in_specs=[
            pl.BlockSpec((ROWS_BLK, D), lambda i: (i, 0)),
            pl.BlockSpec((NC, ROWS_BLK, D), lambda i: (0, i, 0)),
            pl.BlockSpec((D, D), lambda i: (0, 0)),
            pl.BlockSpec((D, D), lambda i: (0, 0)),
        ],
        out_specs=[
            pl.BlockSpec((ROWS_BLK, D), lambda i: (i, 0)),
            pl.BlockSpec((ROWS_BLK, D), lambda i: (i, 0)),
            pl.BlockSpec((ROWS_BLK, D), lambda i: (i, 0)),
        ],
        out_shape=[jax.ShapeDtypeStruct((N_NODE, D), jnp.float32)] * 3,
        interpret=interpret,
    )


def _layer_norm(x, g, b):
    mu = jnp.mean(x, axis=-1, keepdims=True)
    var = jnp.mean((x - mu) * (x - mu), axis=-1, keepdims=True)
    return (x - mu) * lax.rsqrt(var + 1e-6) * g + b


def _tc_final_body(xv_ref, a_ref, xc_ref, g_ref, b_ref, yv_ref, yc_ref):
    g = g_ref[...]
    b = b_ref[...]
    xvn = jax.nn.relu(xv_ref[...] + a_ref[0] + a_ref[1])
    yv_ref[...] = _layer_norm(xvn, g, b)
    yc_ref[...] = _layer_norm(xc_ref[...], g, b)


def _make_tc_final(interpret=False):
    return pl.pallas_call(
        _tc_final_body,
        grid=(N_NODE // ROWS_BLK,),
        in_specs=[
            pl.BlockSpec((ROWS_BLK, D), lambda i: (i, 0)),
            pl.BlockSpec((NC, ROWS_BLK, D), lambda i: (0, i, 0)),
            pl.BlockSpec((ROWS_BLK, D), lambda i: (i, 0)),
            pl.BlockSpec((1, D), lambda i: (0, 0)),
            pl.BlockSpec((1, D), lambda i: (0, 0)),
        ],
        out_specs=[
            pl.BlockSpec((ROWS_BLK, D), lambda i: (i, 0)),
            pl.BlockSpec((ROWS_BLK, D), lambda i: (i, 0)),
        ],
        out_shape=[jax.ShapeDtypeStruct((N_NODE, D), jnp.float32)] * 2,
        interpret=interpret,
    )


def _impl(xv, xc, adj_pos, adj_neg, Wcp, Wcn, Wvp, Wvn, gamma, beta,
          interpret=False):
    sc_segsum = _make_sc_segsum(interpret)
    tc_proj = _make_tc_proj(interpret)
    tc_update = _make_tc_update_proj(interpret)
    tc_final = _make_tc_final(interpret)

    idx_shape = (NC * NS, CPT, K)
    cp = adj_pos[0].astype(jnp.int32).reshape(idx_shape)
    lp = adj_pos[1].astype(jnp.int32).reshape(idx_shape)
    cn = adj_neg[0].astype(jnp.int32).reshape(idx_shape)
    ln_ = adj_neg[1].astype(jnp.int32).reshape(idx_shape)
    g2 = gamma.reshape(1, D)
    b2 = beta.reshape(1, D)

    L = Wcp.shape[0]
    xvp, xvn = tc_proj(xv, Wcp[0], Wcn[0])
    acc_c = sc_segsum(xvp, xvn, lp, cp, ln_, cn)
    xc, xcp, xcn = tc_update(xc, acc_c, Wvp[0], Wvn[0])
    acc_v = sc_segsum(xcp, xcn, cp, lp, cn, ln_)
    for l in range(1, L):
        xv, xvp, xvn = tc_update(xv, acc_v, Wcp[l], Wcn[l])
        acc_c = sc_segsum(xvp, xvn, lp, cp, ln_, cn)
        xc, xcp, xcn = tc_update(xc, acc_c, Wvp[l], Wvn[l])
        acc_v = sc_segsum(xcp, xcn, cp, lp, cn, ln_)
    return tc_final(xv, acc_v, xc, g2, b2)


def kernel(xv, xc, adj_pos, adj_neg, Wcp, Wcn, Wvp, Wvn, gamma, beta):
    yv, yc = _impl(xv, xc, adj_pos, adj_neg, Wcp, Wcn, Wvp, Wvn, gamma, beta)
    return yv, yc


# speedup vs baseline: 9.0455x; 1.0290x over previous
import jax
import jax.numpy as jnp
from jax import lax
from jax.experimental import pallas as pl
from jax.experimental.pallas import tpu as pltpu
from jax.experimental.pallas import tpu_sc as plsc

N_NODE = 10000
D = 128
E = 160000
NC, NS = 2, 16
K = 125
NBUF = 2
CPT = E // (NC * NS * K)
ROWS_PER_TILE = 640
N_PAD = NS * ROWS_PER_TILE
ZROWS = 16


def _sc_segsum_body(xp_hbm, xn_hbm, srcp, dstp, srcn, dstn, out_hbm,
                    zbuf, isrc, idst, rbuf, acc, gsem, ssem, zsem):
    cid = lax.axis_index("c")
    sid = lax.axis_index("s")
    wid = cid * NS + sid

    def _g_start(tab_ref, j, m):
        pltpu.async_copy(tab_ref.at[isrc.at[j]], rbuf.at[m], gsem)

    def _g_wait(tab_ref, j, m):
        pltpu.make_async_copy(tab_ref.at[isrc.at[j]], rbuf.at[m],
                              gsem).wait()

    def _s_start(j, m):
        pltpu.async_copy(rbuf.at[m], acc.at[idst.at[j]], ssem, add=True)

    def _s_wait(j, m):
        pltpu.make_async_copy(rbuf.at[m], acc.at[idst.at[j]], ssem).wait()

    pltpu.sync_copy(srcp.at[wid], isrc)
    pltpu.sync_copy(dstp.at[wid], idst)
    for t in range(NBUF - 1):
        _g_start(xp_hbm, t, t)

    def _z(i, _):
        r = i // 8
        cb = i % 8
        zbuf[r, pl.ds(cb * 16, 16)] = jnp.zeros((16,), jnp.float32)
        return 0
    lax.fori_loop(0, ZROWS * 8, _z, 0)

    def _zacc(i, _):
        pltpu.async_copy(
            zbuf, acc.at[pl.ds(sid * ROWS_PER_TILE + i * ZROWS, ZROWS)], zsem)
        return 0
    lax.fori_loop(0, ROWS_PER_TILE // ZROWS, _zacc, 0)

    def _zwait(i, _):
        pltpu.make_async_copy(
            zbuf, acc.at[pl.ds(sid * ROWS_PER_TILE + i * ZROWS, ZROWS)],
            zsem).wait()
        return 0
    lax.fori_loop(0, ROWS_PER_TILE // ZROWS, _zwait, 0)
    plsc.subcore_barrier()

    for seti, (src_ref, dst_ref, tab_ref) in enumerate(
            ((srcp, dstp, xp_hbm), (srcn, dstn, xn_hbm))):
        if seti > 0:
            pltpu.sync_copy(src_ref.at[wid], isrc)
            pltpu.sync_copy(dst_ref.at[wid], idst)
            for t in range(NBUF - 1):
                _g_start(tab_ref, t, t)

        _g_wait(tab_ref, 0, 0)
        _s_start(0, 0)
        _g_start(tab_ref, NBUF - 1, NBUF - 1)

        def _body(j, _):
            m = j % NBUF
            _g_wait(tab_ref, j, m)
            _s_wait(j - 1, (j - 1) % NBUF)
            _g_start(tab_ref, j + NBUF - 1, (j - 1) % NBUF)
            _s_start(j, m)
            return 0
        lax.fori_loop(1, CPT - NBUF + 1, _body, 0)

        for j in range(CPT - NBUF + 1, CPT):
            _g_wait(tab_ref, j, j % NBUF)
            _s_wait(j - 1, (j - 1) % NBUF)
            _s_start(j, j % NBUF)
        _s_wait(CPT - 1, (CPT - 1) % NBUF)

    plsc.subcore_barrier()
    pltpu.sync_copy(acc.at[pl.ds(sid * ROWS_PER_TILE, ROWS_PER_TILE)],
                    out_hbm.at[cid, pl.ds(sid * ROWS_PER_TILE, ROWS_PER_TILE)])


def _make_sc_segsum(interpret=False):
    mesh = plsc.VectorSubcoreMesh(core_axis_name="c", subcore_axis_name="s",
                                  num_cores=NC, num_subcores=NS)
    return pl.kernel(
        _sc_segsum_body,
        out_type=jax.ShapeDtypeStruct((NC, N_PAD, D), jnp.float32),
        mesh=mesh,
        scratch_types=[
            pltpu.VMEM((ZROWS, D), jnp.float32),
            pltpu.VMEM((CPT, K), jnp.int32),
            pltpu.VMEM((CPT, K), jnp.int32),
            pltpu.VMEM((NBUF, K, D), jnp.float32),
            pltpu.VMEM_SHARED((N_PAD, D), jnp.float32),
            pltpu.SemaphoreType.DMA,
            pltpu.SemaphoreType.DMA,
            pltpu.SemaphoreType.DMA,
        ],
        interpret=interpret,
    )


ROWS_BLK = 2000


def _tc_proj_body(x_ref, w1_ref, w2_ref, p1_ref, p2_ref):
    x = x_ref[...]
    p1_ref[...] = jnp.dot(x, w1_ref[...], preferred_element_type=jnp.float32)
    p2_ref[...] = jnp.dot(x, w2_ref[...], preferred_element_type=jnp.float32)


def _make_tc_proj(interpret=False):
    return pl.pallas_call(
        _tc_proj_body,
        grid=(N_NODE // ROWS_BLK,),
        in_specs=[
            pl.BlockSpec((ROWS_BLK, D), lambda i: (i, 0)),
            pl.BlockSpec((D, D), lambda i: (0, 0)),
            pl.BlockSpec((D, D), lambda i: (0, 0)),
        ],
        out_specs=[
            pl.BlockSpec((ROWS_BLK, D), lambda i: (i, 0)),
            pl.BlockSpec((ROWS_BLK, D), lambda i: (i, 0)),
        ],
        out_shape=[jax.ShapeDtypeStruct((N_NODE, D), jnp.float32)] * 2,
        interpret=interpret,
    )


def _tc_update_proj_body(x_ref, a_ref, w1_ref, w2_ref,
                         xn_ref, p1_ref, p2_ref):
    xn = jax.nn.relu(x_ref[...] + a_ref[0] + a_ref[1])
    xn_ref[...] = xn
    p1_ref[...] = jnp.dot(xn, w1_ref[...], preferred_element_type=jnp.float32)
    p2_ref[...] = jnp.dot(xn, w2_ref[...], preferred_element_type=jnp.float32)


def _make_tc_update_proj(interpret=False):
    return pl.pallas_call(
        _tc_update_proj_body,
        grid=(N_NODE // ROWS_BLK,),
        in_specs=[
            pl.BlockSpec((ROWS_BLK, D), lambda i: (i, 0)),
            pl.BlockSpec((NC, ROWS_BLK, D), lambda i: (0, i, 0)),
            pl.BlockSpec((D, D), lambda i: (0, 0)),
            pl.BlockSpec((D, D), lambda i: (0, 0)),
        ],
        out_specs=[
            pl.BlockSpec((ROWS_BLK, D), lambda i: (i, 0)),
            pl.BlockSpec((ROWS_BLK, D), lambda i: (i, 0)),
            pl.BlockSpec((ROWS_BLK, D), lambda i: (i, 0)),
        ],
        out_shape=[jax.ShapeDtypeStruct((N_NODE, D), jnp.float32)] * 3,
        interpret=interpret,
    )


def _layer_norm(x, g, b):
    mu = jnp.mean(x, axis=-1, keepdims=True)
    var = jnp.mean((x - mu) * (x - mu), axis=-1, keepdims=True)
    return (x - mu) * lax.rsqrt(var + 1e-6) * g + b


def _tc_final_body(xv_ref, a_ref, xc_ref, g_ref, b_ref, yv_ref, yc_ref):
    g = g_ref[...]
    b = b_ref[...]
    xvn = jax.nn.relu(xv_ref[...] + a_ref[0] + a_ref[1])
    yv_ref[...] = _layer_norm(xvn, g, b)
    yc_ref[...] = _layer_norm(xc_ref[...], g, b)


def _make_tc_final(interpret=False):
    return pl.pallas_call(
        _tc_final_body,
        grid=(N_NODE // ROWS_BLK,),
        in_specs=[
            pl.BlockSpec((ROWS_BLK, D), lambda i: (i, 0)),
            pl.BlockSpec((NC, ROWS_BLK, D), lambda i: (0, i, 0)),
            pl.BlockSpec((ROWS_BLK, D), lambda i: (i, 0)),
            pl.BlockSpec((1, D), lambda i: (0, 0)),
            pl.BlockSpec((1, D), lambda i: (0, 0)),
        ],
        out_specs=[
            pl.BlockSpec((ROWS_BLK, D), lambda i: (i, 0)),
            pl.BlockSpec((ROWS_BLK, D), lambda i: (i, 0)),
        ],
        out_shape=[jax.ShapeDtypeStruct((N_NODE, D), jnp.float32)] * 2,
        interpret=interpret,
    )


def _impl(xv, xc, adj_pos, adj_neg, Wcp, Wcn, Wvp, Wvn, gamma, beta,
          interpret=False):
    sc_segsum = _make_sc_segsum(interpret)
    tc_proj = _make_tc_proj(interpret)
    tc_update = _make_tc_update_proj(interpret)
    tc_final = _make_tc_final(interpret)

    idx_shape = (NC * NS, CPT, K)
    cp = adj_pos[0].astype(jnp.int32).reshape(idx_shape)
    lp = adj_pos[1].astype(jnp.int32).reshape(idx_shape)
    cn = adj_neg[0].astype(jnp.int32).reshape(idx_shape)
    ln_ = adj_neg[1].astype(jnp.int32).reshape(idx_shape)
    g2 = gamma.reshape(1, D)
    b2 = beta.reshape(1, D)

    L = Wcp.shape[0]
    xvp, xvn = tc_proj(xv, Wcp[0], Wcn[0])
    acc_c = sc_segsum(xvp, xvn, lp, cp, ln_, cn)
    xc, xcp, xcn = tc_update(xc, acc_c, Wvp[0], Wvn[0])
    acc_v = sc_segsum(xcp, xcn, cp, lp, cn, ln_)
    for l in range(1, L):
        xv, xvp, xvn = tc_update(xv, acc_v, Wcp[l], Wcn[l])
        acc_c = sc_segsum(xvp, xvn, lp, cp, ln_, cn)
        xc, xcp, xcn = tc_update(xc, acc_c, Wvp[l], Wvn[l])
        acc_v = sc_segsum(xcp, xcn, cp, lp, cn, ln_)
    return tc_final(xv, acc_v, xc, g2, b2)


def kernel(xv, xc, adj_pos, adj_neg, Wcp, Wcn, Wvp, Wvn, gamma, beta):
    yv, yc = _impl(xv, xc, adj_pos, adj_neg, Wcp, Wcn, Wvp, Wvn, gamma, beta)
    return yv, yc


# speedup vs baseline: 9.3916x; 1.0383x over previous
import jax
import jax.numpy as jnp
from jax import lax
from jax.experimental import pallas as pl
from jax.experimental.pallas import tpu as pltpu
from jax.experimental.pallas import tpu_sc as plsc

N_NODE = 10000
D = 128
E = 160000
NC, NS = 2, 16
K = 125
NBUF = 2
CPT = E // (NS * K)
CHALF = CPT // 2
ROWS_PER_TILE = 632
N_PAD = NS * ROWS_PER_TILE
ZROWS = 8


def _sc_segsum_body(x_hbm, src_all, dst_all, out_hbm,
                    zbuf, isrc, idst, rbuf, acc, gsem, ssem, zsem):
    cid = lax.axis_index("c")
    sid = lax.axis_index("s")

    def _g_start(j, m):
        pltpu.async_copy(x_hbm.at[isrc.at[j]], rbuf.at[m], gsem)

    def _g_wait(j, m):
        pltpu.make_async_copy(x_hbm.at[isrc.at[j]], rbuf.at[m], gsem).wait()

    def _s_start(j, m):
        pltpu.async_copy(rbuf.at[m], acc.at[idst.at[j]], ssem, add=True)

    def _s_wait(j, m):
        pltpu.make_async_copy(rbuf.at[m], acc.at[idst.at[j]], ssem).wait()

    pltpu.sync_copy(src_all.at[cid, sid, pl.ds(0, CHALF)], isrc)
    pltpu.sync_copy(dst_all.at[cid, sid, pl.ds(0, CHALF)], idst)
    for t in range(NBUF - 1):
        _g_start(t, t)

    def _z(i, _):
        r = i // 8
        cb = i % 8
        zbuf[r, pl.ds(cb * 16, 16)] = jnp.zeros((16,), jnp.float32)
        return 0
    lax.fori_loop(0, ZROWS * 8, _z, 0)

    def _zacc(i, _):
        pltpu.async_copy(
            zbuf, acc.at[pl.ds(sid * ROWS_PER_TILE + i * ZROWS, ZROWS)], zsem)
        return 0
    lax.fori_loop(0, ROWS_PER_TILE // ZROWS, _zacc, 0)

    def _zwait(i, _):
        pltpu.make_async_copy(
            zbuf, acc.at[pl.ds(sid * ROWS_PER_TILE + i * ZROWS, ZROWS)],
            zsem).wait()
        return 0
    lax.fori_loop(0, ROWS_PER_TILE // ZROWS, _zwait, 0)
    plsc.subcore_barrier()

    for half in range(2):
        if half > 0:
            pltpu.sync_copy(src_all.at[cid, sid, pl.ds(CHALF, CHALF)], isrc)
            pltpu.sync_copy(dst_all.at[cid, sid, pl.ds(CHALF, CHALF)], idst)
            for t in range(NBUF - 1):
                _g_start(t, t)

        _g_wait(0, 0)
        _s_start(0, 0)
        _g_start(NBUF - 1, NBUF - 1)

        def _body(j, _):
            m = j % NBUF
            _g_wait(j, m)
            _s_wait(j - 1, (j - 1) % NBUF)
            _g_start(j + NBUF - 1, (j - 1) % NBUF)
            _s_start(j, m)
            return 0
        lax.fori_loop(1, CHALF - NBUF + 1, _body, 0)

        for j in range(CHALF - NBUF + 1, CHALF):
            _g_wait(j, j % NBUF)
            _s_wait(j - 1, (j - 1) % NBUF)
            _s_start(j, j % NBUF)
        _s_wait(CHALF - 1, (CHALF - 1) % NBUF)

    plsc.subcore_barrier()
    pltpu.sync_copy(acc.at[pl.ds(sid * ROWS_PER_TILE, ROWS_PER_TILE)],
                    out_hbm.at[cid, pl.ds(sid * ROWS_PER_TILE, ROWS_PER_TILE)])


def _make_sc_segsum(interpret=False):
    mesh = plsc.VectorSubcoreMesh(core_axis_name="c", subcore_axis_name="s",
                                  num_cores=NC, num_subcores=NS)
    return pl.kernel(
        _sc_segsum_body,
        out_type=jax.ShapeDtypeStruct((NC, N_PAD, D), jnp.float32),
        mesh=mesh,
        scratch_types=[
            pltpu.VMEM((ZROWS, D), jnp.float32),
            pltpu.VMEM((CHALF, K), jnp.int32),
            pltpu.VMEM((CHALF, K), jnp.int32),
            pltpu.VMEM((NBUF, K, D), jnp.float32),
            pltpu.VMEM_SHARED((N_PAD, D), jnp.float32),
            pltpu.SemaphoreType.DMA,
            pltpu.SemaphoreType.DMA,
            pltpu.SemaphoreType.DMA,
        ],
        interpret=interpret,
    )


ROWS_BLK = 2000


def _tc_update_body(x_ref, s_ref, w1_ref, w2_ref, xn_ref):
    msg = jnp.dot(s_ref[0], w1_ref[...], preferred_element_type=jnp.float32)
    msg += jnp.dot(s_ref[1], w2_ref[...], preferred_element_type=jnp.float32)
    xn_ref[...] = jax.nn.relu(x_ref[...] + msg)


def _make_tc_update(interpret=False):
    return pl.pallas_call(
        _tc_update_body,
        grid=(N_NODE // ROWS_BLK,),
        in_specs=[
            pl.BlockSpec((ROWS_BLK, D), lambda i: (i, 0)),
            pl.BlockSpec((NC, ROWS_BLK, D), lambda i: (0, i, 0)),
            pl.BlockSpec((D, D), lambda i: (0, 0)),
            pl.BlockSpec((D, D), lambda i: (0, 0)),
        ],
        out_specs=pl.BlockSpec((ROWS_BLK, D), lambda i: (i, 0)),
        out_shape=jax.ShapeDtypeStruct((N_NODE, D), jnp.float32),
        interpret=interpret,
    )


def _layer_norm(x, g, b):
    mu = jnp.mean(x, axis=-1, keepdims=True)
    var = jnp.mean((x - mu) * (x - mu), axis=-1, keepdims=True)
    return (x - mu) * lax.rsqrt(var + 1e-6) * g + b


def _tc_final_body(xv_ref, s_ref, w1_ref, w2_ref, xc_ref, g_ref, b_ref,
                   yv_ref, yc_ref):
    g = g_ref[...]
    b = b_ref[...]
    msg = jnp.dot(s_ref[0], w1_ref[...], preferred_element_type=jnp.float32)
    msg += jnp.dot(s_ref[1], w2_ref[...], preferred_element_type=jnp.float32)
    xvn = jax.nn.relu(xv_ref[...] + msg)
    yv_ref[...] = _layer_norm(xvn, g, b)
    yc_ref[...] = _layer_norm(xc_ref[...], g, b)


def _make_tc_final(interpret=False):
    return pl.pallas_call(
        _tc_final_body,
        grid=(N_NODE // ROWS_BLK,),
        in_specs=[
            pl.BlockSpec((ROWS_BLK, D), lambda i: (i, 0)),
            pl.BlockSpec((NC, ROWS_BLK, D), lambda i: (0, i, 0)),
            pl.BlockSpec((D, D), lambda i: (0, 0)),
            pl.BlockSpec((D, D), lambda i: (0, 0)),
            pl.BlockSpec((ROWS_BLK, D), lambda i: (i, 0)),
            pl.BlockSpec((1, D), lambda i: (0, 0)),
            pl.BlockSpec((1, D), lambda i: (0, 0)),
        ],
        out_specs=[
            pl.BlockSpec((ROWS_BLK, D), lambda i: (i, 0)),
            pl.BlockSpec((ROWS_BLK, D), lambda i: (i, 0)),
        ],
        out_shape=[jax.ShapeDtypeStruct((N_NODE, D), jnp.float32)] * 2,
        interpret=interpret,
    )


def _impl(xv, xc, adj_pos, adj_neg, Wcp, Wcn, Wvp, Wvn, gamma, beta,
          interpret=False):
    sc_segsum = _make_sc_segsum(interpret)
    tc_update = _make_tc_update(interpret)
    tc_final = _make_tc_final(interpret)

    idx_shape = (NC, NS, CPT, K)
    cls_idx = jnp.stack([adj_pos[0], adj_neg[0]]).astype(jnp.int32)
    lit_idx = jnp.stack([adj_pos[1], adj_neg[1]]).astype(jnp.int32)
    cls_idx = cls_idx.reshape(idx_shape)
    lit_idx = lit_idx.reshape(idx_shape)
    g2 = gamma.reshape(1, D)
    b2 = beta.reshape(1, D)

    L = Wcp.shape[0]
    for l in range(L):
        s_c = sc_segsum(xv, lit_idx, cls_idx)
        xc = tc_update(xc, s_c, Wcp[l], Wcn[l])
        s_v = sc_segsum(xc, cls_idx, lit_idx)
        if l < L - 1:
            xv = tc_update(xv, s_v, Wvp[l], Wvn[l])
    return tc_final(xv, s_v, Wvp[L - 1], Wvn[L - 1], xc, g2, b2)


def kernel(xv, xc, adj_pos, adj_neg, Wcp, Wcn, Wvp, Wvn, gamma, beta):
    yv, yc = _impl(xv, xc, adj_pos, adj_neg, Wcp, Wcn, Wvp, Wvn, gamma, beta)
    return yv, yc


# speedup vs baseline: 9.3968x; 1.0005x over previous
import jax
import jax.numpy as jnp
from jax import lax
from jax.experimental import pallas as pl
from jax.experimental.pallas import tpu as pltpu
from jax.experimental.pallas import tpu_sc as plsc

N_NODE = 10000
D = 128
E = 160000
NC, NS = 2, 16
K = 125
NBUF = 2
CPT = E // (NS * K)
CHALF = CPT // 2
ROWS_PER_TILE = 632
N_PAD = NS * ROWS_PER_TILE
ZROWS = 8


def _sc_segsum_body(x_hbm, src_all, dst_all, out_hbm,
                    zbuf, isrc, idst, rbuf, acc, gsem, ssem, zsem):
    cid = lax.axis_index("c")
    sid = lax.axis_index("s")

    def _g_start(j, m):
        pltpu.async_copy(x_hbm.at[isrc.at[j]], rbuf.at[m], gsem)

    def _g_wait(j, m):
        pltpu.make_async_copy(x_hbm.at[isrc.at[j]], rbuf.at[m], gsem).wait()

    def _s_start(j, m):
        pltpu.async_copy(rbuf.at[m], acc.at[idst.at[j]], ssem, add=True)

    def _s_wait(j, m):
        pltpu.make_async_copy(rbuf.at[m], acc.at[idst.at[j]], ssem).wait()

    pltpu.sync_copy(src_all.at[cid, sid, pl.ds(0, CHALF)], isrc)
    pltpu.sync_copy(dst_all.at[cid, sid, pl.ds(0, CHALF)], idst)
    for t in range(NBUF - 1):
        _g_start(t, t)

    def _z(i, _):
        r = i // 8
        cb = i % 8
        zbuf[r, pl.ds(cb * 16, 16)] = jnp.zeros((16,), jnp.float32)
        return 0
    lax.fori_loop(0, ZROWS * 8, _z, 0)

    def _zacc(i, _):
        pltpu.async_copy(
            zbuf, acc.at[pl.ds(sid * ROWS_PER_TILE + i * ZROWS, ZROWS)], zsem)
        return 0
    lax.fori_loop(0, ROWS_PER_TILE // ZROWS, _zacc, 0)

    def _zwait(i, _):
        pltpu.make_async_copy(
            zbuf, acc.at[pl.ds(sid * ROWS_PER_TILE + i * ZROWS, ZROWS)],
            zsem).wait()
        return 0
    lax.fori_loop(0, ROWS_PER_TILE // ZROWS, _zwait, 0)
    plsc.subcore_barrier()

    for half in range(2):
        if half > 0:
            pltpu.sync_copy(src_all.at[cid, sid, pl.ds(CHALF, CHALF)], isrc)
            pltpu.sync_copy(dst_all.at[cid, sid, pl.ds(CHALF, CHALF)], idst)
            for t in range(NBUF - 1):
                _g_start(t, t)

        _g_wait(0, 0)
        _s_start(0, 0)
        _g_start(NBUF - 1, NBUF - 1)

        def _body(t, _):
            j = 2 * t + 1
            _g_wait(j, 1)
            _s_wait(j - 1, 0)
            _g_start(j + 1, 0)
            _s_start(j, 1)
            _g_wait(j + 1, 0)
            _s_wait(j, 1)
            _g_start(j + 2, 1)
            _s_start(j + 1, 0)
            return 0
        lax.fori_loop(0, (CHALF - NBUF) // 2, _body, 0)

        for j in range(CHALF - NBUF + 1, CHALF):
            _g_wait(j, j % NBUF)
            _s_wait(j - 1, (j - 1) % NBUF)
            _s_start(j, j % NBUF)
        _s_wait(CHALF - 1, (CHALF - 1) % NBUF)

    plsc.subcore_barrier()
    pltpu.sync_copy(acc.at[pl.ds(sid * ROWS_PER_TILE, ROWS_PER_TILE)],
                    out_hbm.at[cid, pl.ds(sid * ROWS_PER_TILE, ROWS_PER_TILE)])


def _make_sc_segsum(interpret=False):
    mesh = plsc.VectorSubcoreMesh(core_axis_name="c", subcore_axis_name="s",
                                  num_cores=NC, num_subcores=NS)
    return pl.kernel(
        _sc_segsum_body,
        out_type=jax.ShapeDtypeStruct((NC, N_PAD, D), jnp.float32),
        mesh=mesh,
        scratch_types=[
            pltpu.VMEM((ZROWS, D), jnp.float32),
            pltpu.VMEM((CHALF, K), jnp.int32),
            pltpu.VMEM((CHALF, K), jnp.int32),
            pltpu.VMEM((NBUF, K, D), jnp.float32),
            pltpu.VMEM_SHARED((N_PAD, D), jnp.float32),
            pltpu.SemaphoreType.DMA,
            pltpu.SemaphoreType.DMA,
            pltpu.SemaphoreType.DMA,
        ],
        interpret=interpret,
    )


ROWS_BLK = 2000


def _tc_update_body(x_ref, s_ref, w1_ref, w2_ref, xn_ref):
    msg = jnp.dot(s_ref[0], w1_ref[...], preferred_element_type=jnp.float32)
    msg += jnp.dot(s_ref[1], w2_ref[...], preferred_element_type=jnp.float32)
    xn_ref[...] = jax.nn.relu(x_ref[...] + msg)


def _make_tc_update(interpret=False):
    return pl.pallas_call(
        _tc_update_body,
        grid=(N_NODE // ROWS_BLK,),
        in_specs=[
            pl.BlockSpec((ROWS_BLK, D), lambda i: (i, 0)),
            pl.BlockSpec((NC, ROWS_BLK, D), lambda i: (0, i, 0)),
            pl.BlockSpec((D, D), lambda i: (0, 0)),
            pl.BlockSpec((D, D), lambda i: (0, 0)),
        ],
        out_specs=pl.BlockSpec((ROWS_BLK, D), lambda i: (i, 0)),
        out_shape=jax.ShapeDtypeStruct((N_NODE, D), jnp.float32),
        interpret=interpret,
    )


def _layer_norm(x, g, b):
    mu = jnp.mean(x, axis=-1, keepdims=True)
    var = jnp.mean((x - mu) * (x - mu), axis=-1, keepdims=True)
    return (x - mu) * lax.rsqrt(var + 1e-6) * g + b


def _tc_final_body(xv_ref, s_ref, w1_ref, w2_ref, xc_ref, g_ref, b_ref,
                   yv_ref, yc_ref):
    g = g_ref[...]
    b = b_ref[...]
    msg = jnp.dot(s_ref[0], w1_ref[...], preferred_element_type=jnp.float32)
    msg += jnp.dot(s_ref[1], w2_ref[...], preferred_element_type=jnp.float32)
    xvn = jax.nn.relu(xv_ref[...] + msg)
    yv_ref[...] = _layer_norm(xvn, g, b)
    yc_ref[...] = _layer_norm(xc_ref[...], g, b)


def _make_tc_final(interpret=False):
    return pl.pallas_call(
        _tc_final_body,
        grid=(N_NODE // ROWS_BLK,),
        in_specs=[
            pl.BlockSpec((ROWS_BLK, D), lambda i: (i, 0)),
            pl.BlockSpec((NC, ROWS_BLK, D), lambda i: (0, i, 0)),
            pl.BlockSpec((D, D), lambda i: (0, 0)),
            pl.BlockSpec((D, D), lambda i: (0, 0)),
            pl.BlockSpec((ROWS_BLK, D), lambda i: (i, 0)),
            pl.BlockSpec((1, D), lambda i: (0, 0)),
            pl.BlockSpec((1, D), lambda i: (0, 0)),
        ],
        out_specs=[
            pl.BlockSpec((ROWS_BLK, D), lambda i: (i, 0)),
            pl.BlockSpec((ROWS_BLK, D), lambda i: (i, 0)),
        ],
        out_shape=[jax.ShapeDtypeStruct((N_NODE, D), jnp.float32)] * 2,
        interpret=interpret,
    )


def _impl(xv, xc, adj_pos, adj_neg, Wcp, Wcn, Wvp, Wvn, gamma, beta,
          interpret=False):
    sc_segsum = _make_sc_segsum(interpret)
    tc_update = _make_tc_update(interpret)
    tc_final = _make_tc_final(interpret)

    idx_shape = (NC, NS, CPT, K)
    cls_idx = jnp.stack([adj_pos[0], adj_neg[0]]).astype(jnp.int32)
    lit_idx = jnp.stack([adj_pos[1], adj_neg[1]]).astype(jnp.int32)
    cls_idx = cls_idx.reshape(idx_shape)
    lit_idx = lit_idx.reshape(idx_shape)
    g2 = gamma.reshape(1, D)
    b2 = beta.reshape(1, D)

    L = Wcp.shape[0]
    for l in range(L):
        s_c = sc_segsum(xv, lit_idx, cls_idx)
        xc = tc_update(xc, s_c, Wcp[l], Wcn[l])
        s_v = sc_segsum(xc, cls_idx, lit_idx)
        if l < L - 1:
            xv = tc_update(xv, s_v, Wvp[l], Wvn[l])
    return tc_final(xv, s_v, Wvp[L - 1], Wvn[L - 1], xc, g2, b2)


def kernel(xv, xc, adj_pos, adj_neg, Wcp, Wcn, Wvp, Wvn, gamma, beta):
    yv, yc = _impl(xv, xc, adj_pos, adj_neg, Wcp, Wcn, Wvp, Wvn, gamma, beta)
    return yv, yc
